# Initial kernel scaffold; baseline (speedup 1.0000x reference)
import jax
import jax.numpy as jnp
from jax.experimental import pallas as pl


def kernel(left_embeddings, right_embeddings, left_n_tokens, right_n_tokens, W_tok_n, b_tok_n, W_tok_g, b_tok_g, W_tok_lin, b_tok_lin, attr_emb_left, attr_emb_right, W_ent_n, b_ent_n, W_ent_g, b_ent_g, W_ent_lin, b_ent_lin, empty_attr_res):
    raise NotImplementedError("write your pallas kernel here")


# speedup vs baseline: 2.3767x; 2.3767x over previous
import functools

import jax
import jax.numpy as jnp
from jax.experimental import pallas as pl
from jax.experimental.pallas import tpu as pltpu

D = 256
L = 256
R = 256
NATTR = 4
SEG = L // NATTR
TL = 8
NT = L // TL
ENT = 2 * NATTR * D
ETILE = 256
ENT_NT = ENT // ETILE


def _token_kernel(lt_ref, lf_ref, rf_ref, wn_ref, bn_ref, wg_ref, bg_ref,
                  wlt_ref, ael_ref, aer_ref, empty_ref, ln_ref, rn_ref,
                  xent_ref, lcm_ref, bestv_ref, besti_ref):
    i = pl.program_id(0)
    lt = lt_ref[...]
    rt = rf_ref[...]

    x3 = jnp.abs(lt[:, None, :] - rt[None, :, :])
    x = x3.reshape(TL * R, D)

    a = jnp.dot(x, wn_ref[...], preferred_element_type=jnp.float32)
    h = jax.nn.relu(a + bn_ref[...])
    g = jax.nn.sigmoid(
        jnp.dot(x, wg_ref[...], preferred_element_type=jnp.float32)
        + bg_ref[...])
    hw = h * g + x - g * x
    s = jnp.sum(hw * wlt_ref[...], axis=1).reshape(TL, R)

    iota_r = jax.lax.broadcasted_iota(jnp.int32, (TL, R), 1)
    smax_r = jnp.max(s, axis=1, keepdims=True)
    idx_r = jnp.min(jnp.where(s == smax_r, iota_r, R), axis=1, keepdims=True)

    onehot = (iota_r == idx_r).astype(jnp.float32)
    picked = jnp.dot(onehot, rt, preferred_element_type=jnp.float32)
    lcm_ref[pl.ds(i * TL, TL), :] = jnp.abs(lt - picked)

    iota_l = jax.lax.broadcasted_iota(jnp.int32, (TL, R), 0)
    smax_l = jnp.max(s, axis=0, keepdims=True)
    cand = jnp.min(jnp.where(s == smax_l, iota_l, TL), axis=0,
                   keepdims=True) + i * TL

    @pl.when(i == 0)
    def _init():
        bestv_ref[...] = smax_l
        besti_ref[...] = cand

    @pl.when(i > 0)
    def _update():
        upd = smax_l > bestv_ref[...]
        bestv_ref[...] = jnp.where(upd, smax_l, bestv_ref[...])
        besti_ref[...] = jnp.where(upd, cand, besti_ref[...])

    @pl.when(i == NT - 1)
    def _finalize():
        lf = lf_ref[...]
        bi = besti_ref[...].reshape(R, 1)
        iota_rl = jax.lax.broadcasted_iota(jnp.int32, (R, L), 1)
        oh_l = (iota_rl == bi).astype(jnp.float32)
        rcm = jnp.abs(rt - jnp.dot(oh_l, lf,
                                   preferred_element_type=jnp.float32))

        def attr_rows(tok_all, emb_ref, cm, n_ref, base):
            for att in range(NATTR):
                tok = tok_all[att * SEG:(att + 1) * SEG, :]
                logits = jnp.sum(tok * emb_ref[att:att + 1, :], axis=1,
                                 keepdims=True)
                e = jnp.exp(logits - jnp.max(logits))
                w = e / jnp.sum(e)
                seg = cm[att * SEG:(att + 1) * SEG, :]
                val = jnp.sum(w * seg, axis=0, keepdims=True)
                val = jnp.where(n_ref[att] == 0, empty_ref[...], val)
                xent_ref[:, pl.ds((base + att) * D, D)] = val

        attr_rows(lf, ael_ref, lcm_ref[...], ln_ref, 0)
        attr_rows(rt, aer_ref, rcm, rn_ref, NATTR)


def _entity_kernel(x_ref, xb_ref, wn_ref, bn_ref, wg_ref, bg_ref,
                   wl_ref, bl_ref, out_ref):
    j = pl.program_id(0)
    xa = x_ref[...]
    h = jax.nn.relu(
        jnp.dot(xa, wn_ref[...], preferred_element_type=jnp.float32)
        + bn_ref[...])
    g = jax.nn.sigmoid(
        jnp.dot(xa, wg_ref[...], preferred_element_type=jnp.float32)
        + bg_ref[...])
    xb = xb_ref[...]
    hw = h * g + xb - g * xb

    @pl.when(j == 0)
    def _init():
        out_ref[...] = bl_ref[...]

    out_ref[...] += jnp.dot(hw, wl_ref[...],
                            preferred_element_type=jnp.float32)

    @pl.when(j == ENT_NT - 1)
    def _softmax():
        v = out_ref[...]
        e = jnp.exp(v - jnp.max(v))
        out_ref[...] = e / jnp.sum(e)


@functools.partial(jax.jit, static_argnames=("interpret",))
def _run(left, right, ln, rn, wtn, btn, wtg, btg, wtl, ael, aer,
         wen, ben, weg, beg, wel, bel, empty, interpret=False):
    f32 = jnp.float32
    const = lambda shape: pl.BlockSpec(shape, lambda i: (0, 0))
    xent = pl.pallas_call(
        _token_kernel,
        grid=(NT,),
        in_specs=[
            pl.BlockSpec((TL, D), lambda i: (i, 0)),
            const((L, D)),
            const((R, D)),
            const((D, D)), const((1, D)),
            const((D, D)), const((1, D)),
            const((1, D)),
            const((NATTR, D)), const((NATTR, D)),
            const((1, D)),
            pl.BlockSpec(memory_space=pltpu.SMEM),
            pl.BlockSpec(memory_space=pltpu.SMEM),
        ],
        out_specs=const((1, ENT)),
        out_shape=jax.ShapeDtypeStruct((1, ENT), f32),
        scratch_shapes=[
            pltpu.VMEM((L, D), f32),
            pltpu.VMEM((1, R), f32),
            pltpu.VMEM((1, R), jnp.int32),
        ],
        interpret=interpret,
    )(left, left, right, wtn, btn.reshape(1, D), wtg, btg.reshape(1, D),
      wtl.reshape(1, D), ael, aer, empty.reshape(1, D), ln, rn)

    out = pl.pallas_call(
        _entity_kernel,
        grid=(ENT_NT,),
        in_specs=[
            const((1, ENT)),
            pl.BlockSpec((1, ETILE), lambda j: (0, j)),
            pl.BlockSpec((ENT, ETILE), lambda j: (0, j)),
            pl.BlockSpec((1, ETILE), lambda j: (0, j)),
            pl.BlockSpec((ENT, ETILE), lambda j: (0, j)),
            pl.BlockSpec((1, ETILE), lambda j: (0, j)),
            pl.BlockSpec((ETILE, 2), lambda j: (j, 0)),
            pl.BlockSpec((1, 2), lambda j: (0, 0)),
        ],
        out_specs=const((1, 2)),
        out_shape=jax.ShapeDtypeStruct((1, 2), f32),
        interpret=interpret,
    )(xent, xent, wen, ben.reshape(1, ENT), weg, beg.reshape(1, ENT),
      wel, bel.reshape(1, 2))
    return out.reshape(-1)


def kernel(left_embeddings, right_embeddings, left_n_tokens, right_n_tokens,
           W_tok_n, b_tok_n, W_tok_g, b_tok_g, W_tok_lin, b_tok_lin,
           attr_emb_left, attr_emb_right, W_ent_n, b_ent_n, W_ent_g, b_ent_g,
           W_ent_lin, b_ent_lin, empty_attr_res):
    return _run(left_embeddings, right_embeddings, left_n_tokens,
                right_n_tokens, W_tok_n, b_tok_n, W_tok_g, b_tok_g,
                W_tok_lin.reshape(D), attr_emb_left, attr_emb_right,
                W_ent_n, b_ent_n, W_ent_g, b_ent_g, W_ent_lin, b_ent_lin,
                empty_attr_res)


# speedup vs baseline: 3.5777x; 1.5053x over previous
import functools

import jax
import jax.numpy as jnp
from jax.experimental import pallas as pl
from jax.experimental.pallas import tpu as pltpu

D = 256
L = 256
R = 256
NATTR = 4
SEG = L // NATTR
TL = 8
NT = L // TL
ENT = 2 * NATTR * D
ETILE = 256
ENT_NT = ENT // ETILE


def _token_kernel(lt_ref, lf_ref, rf_ref, wall_ref, ball_ref,
                  wlt_ref, ael_ref, aer_ref, empty_ref, ln_ref, rn_ref,
                  xent_ref, s_ref):
    i = pl.program_id(0)
    lt = lt_ref[...]
    rt = rf_ref[...]

    x3 = jnp.abs(lt[:, None, :] - rt[None, :, :])
    x = x3.reshape(TL * R, D)

    y = jnp.dot(x, wall_ref[...], preferred_element_type=jnp.float32)
    yb = y + ball_ref[...]
    h = jax.nn.relu(yb[:, :D])
    g = jax.nn.sigmoid(yb[:, D:])
    hw = g * (h - x) + x
    s_ref[pl.ds(i * TL, TL), :] = jnp.sum(hw * wlt_ref[...],
                                          axis=1).reshape(TL, R)

    @pl.when(i == NT - 1)
    def _finalize():
        lf = lf_ref[...]
        S = s_ref[...]

        iota_lr = jax.lax.broadcasted_iota(jnp.int32, (L, R), 1)
        mx = jnp.max(S, axis=1, keepdims=True)
        idx = jnp.min(jnp.where(S == mx, iota_lr, R), axis=1, keepdims=True)
        ohl = (iota_lr == idx).astype(jnp.float32)
        lcm = jnp.abs(lf - jnp.dot(ohl, rt,
                                   preferred_element_type=jnp.float32))

        St = S.T
        iota_rl = jax.lax.broadcasted_iota(jnp.int32, (R, L), 1)
        mx2 = jnp.max(St, axis=1, keepdims=True)
        idx2 = jnp.min(jnp.where(St == mx2, iota_rl, L), axis=1,
                       keepdims=True)
        ohr = (iota_rl == idx2).astype(jnp.float32)
        rcm = jnp.abs(rt - jnp.dot(ohr, lf,
                                   preferred_element_type=jnp.float32))

        def attr_rows(tok_all, emb_ref, cm, n_ref, base):
            for att in range(NATTR):
                tok = tok_all[att * SEG:(att + 1) * SEG, :]
                logits = jnp.sum(tok * emb_ref[att:att + 1, :], axis=1,
                                 keepdims=True)
                e = jnp.exp(logits - jnp.max(logits))
                w = e / jnp.sum(e)
                seg = cm[att * SEG:(att + 1) * SEG, :]
                val = jnp.sum(w * seg, axis=0, keepdims=True)
                val = jnp.where(n_ref[att] == 0, empty_ref[...], val)
                xent_ref[:, pl.ds((base + att) * D, D)] = val

        attr_rows(lf, ael_ref, lcm, ln_ref, 0)
        attr_rows(rt, aer_ref, rcm, rn_ref, NATTR)


def _entity_kernel(x_ref, xb_ref, wn_ref, bn_ref, wg_ref, bg_ref,
                   wl_ref, bl_ref, out_ref):
    j = pl.program_id(0)
    xa = x_ref[...]
    h = jax.nn.relu(
        jnp.dot(xa, wn_ref[...], preferred_element_type=jnp.float32)
        + bn_ref[...])
    g = jax.nn.sigmoid(
        jnp.dot(xa, wg_ref[...], preferred_element_type=jnp.float32)
        + bg_ref[...])
    xb = xb_ref[...]
    hw = h * g + xb - g * xb

    @pl.when(j == 0)
    def _init():
        out_ref[...] = bl_ref[...]

    out_ref[...] += jnp.dot(hw, wl_ref[...],
                            preferred_element_type=jnp.float32)

    @pl.when(j == ENT_NT - 1)
    def _softmax():
        v = out_ref[...]
        e = jnp.exp(v - jnp.max(v))
        out_ref[...] = e / jnp.sum(e)


@functools.partial(jax.jit, static_argnames=("interpret",))
def _run(left, right, ln, rn, wtn, btn, wtg, btg, wtl, ael, aer,
         wen, ben, weg, beg, wel, bel, empty, interpret=False):
    f32 = jnp.float32
    const = lambda shape: pl.BlockSpec(shape, lambda i: (0, 0))
    xent = pl.pallas_call(
        _token_kernel,
        grid=(NT,),
        in_specs=[
            pl.BlockSpec((TL, D), lambda i: (i, 0)),
            const((L, D)),
            const((R, D)),
            const((D, 2 * D)),
            const((1, 2 * D)),
            const((1, D)),
            const((NATTR, D)), const((NATTR, D)),
            const((1, D)),
            pl.BlockSpec(memory_space=pltpu.SMEM),
            pl.BlockSpec(memory_space=pltpu.SMEM),
        ],
        out_specs=const((1, ENT)),
        out_shape=jax.ShapeDtypeStruct((1, ENT), f32),
        scratch_shapes=[
            pltpu.VMEM((L, R), f32),
        ],
        interpret=interpret,
    )(left, left, right,
      jnp.concatenate([wtn, wtg], axis=1),
      jnp.concatenate([btn, btg]).reshape(1, 2 * D),
      wtl.reshape(1, D), ael, aer, empty.reshape(1, D), ln, rn)

    out = pl.pallas_call(
        _entity_kernel,
        grid=(ENT_NT,),
        in_specs=[
            const((1, ENT)),
            pl.BlockSpec((1, ETILE), lambda j: (0, j)),
            pl.BlockSpec((ENT, ETILE), lambda j: (0, j)),
            pl.BlockSpec((1, ETILE), lambda j: (0, j)),
            pl.BlockSpec((ENT, ETILE), lambda j: (0, j)),
            pl.BlockSpec((1, ETILE), lambda j: (0, j)),
            pl.BlockSpec((ETILE, 2), lambda j: (j, 0)),
            pl.BlockSpec((1, 2), lambda j: (0, 0)),
        ],
        out_specs=const((1, 2)),
        out_shape=jax.ShapeDtypeStruct((1, 2), f32),
        interpret=interpret,
    )(xent, xent, wen, ben.reshape(1, ENT), weg, beg.reshape(1, ENT),
      wel, bel.reshape(1, 2))
    return out.reshape(-1)


def kernel(left_embeddings, right_embeddings, left_n_tokens, right_n_tokens,
           W_tok_n, b_tok_n, W_tok_g, b_tok_g, W_tok_lin, b_tok_lin,
           attr_emb_left, attr_emb_right, W_ent_n, b_ent_n, W_ent_g, b_ent_g,
           W_ent_lin, b_ent_lin, empty_attr_res):
    return _run(left_embeddings, right_embeddings, left_n_tokens,
                right_n_tokens, W_tok_n, b_tok_n, W_tok_g, b_tok_g,
                W_tok_lin.reshape(D), attr_emb_left, attr_emb_right,
                W_ent_n, b_ent_n, W_ent_g, b_ent_g, W_ent_lin, b_ent_lin,
                empty_attr_res)


# speedup vs baseline: 4.0981x; 1.1455x over previous
import functools

import jax
import jax.numpy as jnp
from jax.experimental import pallas as pl
from jax.experimental.pallas import tpu as pltpu

D = 256
L = 256
R = 256
NATTR = 4
SEG = L // NATTR
TL = 32
NT = L // TL
NCHUNK = 4
CROWS = TL // NCHUNK
ENT = 2 * NATTR * D
ETILE = 256
ENT_NT = ENT // ETILE


def _token_kernel(lt_ref, lf_ref, rf_ref, wall_ref, ball_ref,
                  wlt_ref, ael_ref, aer_ref, empty_ref, ln_ref, rn_ref,
                  xent_ref, s_ref):
    i = pl.program_id(0)
    rt = rf_ref[...]

    for c in range(NCHUNK):
        lt = lt_ref[c * CROWS:(c + 1) * CROWS, :]
        x3 = jnp.abs(lt[:, None, :] - rt[None, :, :])
        x = x3.reshape(CROWS * R, D)
        y = jnp.dot(x, wall_ref[...], preferred_element_type=jnp.float32)
        yb = y + ball_ref[...]
        h = jax.nn.relu(yb[:, :D])
        g = jax.nn.sigmoid(yb[:, D:])
        hw = g * (h - x) + x
        s_ref[pl.ds(i * TL + c * CROWS, CROWS), :] = jnp.sum(
            hw * wlt_ref[...], axis=1).reshape(CROWS, R)

    @pl.when(i == NT - 1)
    def _finalize():
        lf = lf_ref[...]
        S = s_ref[...]

        iota_lr = jax.lax.broadcasted_iota(jnp.int32, (L, R), 1)
        mx = jnp.max(S, axis=1, keepdims=True)
        idx = jnp.min(jnp.where(S == mx, iota_lr, R), axis=1, keepdims=True)
        ohl = (iota_lr == idx).astype(jnp.float32)
        lcm = jnp.abs(lf - jnp.dot(ohl, rt,
                                   preferred_element_type=jnp.float32))

        St = S.T
        iota_rl = jax.lax.broadcasted_iota(jnp.int32, (R, L), 1)
        mx2 = jnp.max(St, axis=1, keepdims=True)
        idx2 = jnp.min(jnp.where(St == mx2, iota_rl, L), axis=1,
                       keepdims=True)
        ohr = (iota_rl == idx2).astype(jnp.float32)
        rcm = jnp.abs(rt - jnp.dot(ohr, lf,
                                   preferred_element_type=jnp.float32))

        def attr_rows(tok_all, emb_ref, cm, n_ref, base):
            for att in range(NATTR):
                tok = tok_all[att * SEG:(att + 1) * SEG, :]
                logits = jnp.sum(tok * emb_ref[att:att + 1, :], axis=1,
                                 keepdims=True)
                e = jnp.exp(logits - jnp.max(logits))
                w = e / jnp.sum(e)
                seg = cm[att * SEG:(att + 1) * SEG, :]
                val = jnp.sum(w * seg, axis=0, keepdims=True)
                val = jnp.where(n_ref[att] == 0, empty_ref[...], val)
                xent_ref[:, pl.ds((base + att) * D, D)] = val

        attr_rows(lf, ael_ref, lcm, ln_ref, 0)
        attr_rows(rt, aer_ref, rcm, rn_ref, NATTR)


def _entity_kernel(x_ref, xb_ref, wn_ref, bn_ref, wg_ref, bg_ref,
                   wl_ref, bl_ref, out_ref):
    j = pl.program_id(0)
    xa = x_ref[...]
    h = jax.nn.relu(
        jnp.dot(xa, wn_ref[...], preferred_element_type=jnp.float32)
        + bn_ref[...])
    g = jax.nn.sigmoid(
        jnp.dot(xa, wg_ref[...], preferred_element_type=jnp.float32)
        + bg_ref[...])
    xb = xb_ref[...]
    hw = h * g + xb - g * xb

    @pl.when(j == 0)
    def _init():
        out_ref[...] = bl_ref[...]

    out_ref[...] += jnp.dot(hw, wl_ref[...],
                            preferred_element_type=jnp.float32)

    @pl.when(j == ENT_NT - 1)
    def _softmax():
        v = out_ref[...]
        e = jnp.exp(v - jnp.max(v))
        out_ref[...] = e / jnp.sum(e)


@functools.partial(jax.jit, static_argnames=("interpret",))
def _run(left, right, ln, rn, wtn, btn, wtg, btg, wtl, ael, aer,
         wen, ben, weg, beg, wel, bel, empty, interpret=False):
    f32 = jnp.float32
    const = lambda shape: pl.BlockSpec(shape, lambda i: (0, 0))
    xent = pl.pallas_call(
        _token_kernel,
        grid=(NT,),
        in_specs=[
            pl.BlockSpec((TL, D), lambda i: (i, 0)),
            const((L, D)),
            const((R, D)),
            const((D, 2 * D)),
            const((1, 2 * D)),
            const((1, D)),
            const((NATTR, D)), const((NATTR, D)),
            const((1, D)),
            pl.BlockSpec(memory_space=pltpu.SMEM),
            pl.BlockSpec(memory_space=pltpu.SMEM),
        ],
        out_specs=const((1, ENT)),
        out_shape=jax.ShapeDtypeStruct((1, ENT), f32),
        scratch_shapes=[
            pltpu.VMEM((L, R), f32),
        ],
        interpret=interpret,
    )(left, left, right,
      jnp.concatenate([wtn, wtg], axis=1),
      jnp.concatenate([btn, btg]).reshape(1, 2 * D),
      wtl.reshape(1, D), ael, aer, empty.reshape(1, D), ln, rn)

    out = pl.pallas_call(
        _entity_kernel,
        grid=(ENT_NT,),
        in_specs=[
            const((1, ENT)),
            pl.BlockSpec((1, ETILE), lambda j: (0, j)),
            pl.BlockSpec((ENT, ETILE), lambda j: (0, j)),
            pl.BlockSpec((1, ETILE), lambda j: (0, j)),
            pl.BlockSpec((ENT, ETILE), lambda j: (0, j)),
            pl.BlockSpec((1, ETILE), lambda j: (0, j)),
            pl.BlockSpec((ETILE, 2), lambda j: (j, 0)),
            pl.BlockSpec((1, 2), lambda j: (0, 0)),
        ],
        out_specs=const((1, 2)),
        out_shape=jax.ShapeDtypeStruct((1, 2), f32),
        interpret=interpret,
    )(xent, xent, wen, ben.reshape(1, ENT), weg, beg.reshape(1, ENT),
      wel, bel.reshape(1, 2))
    return out.reshape(-1)


def kernel(left_embeddings, right_embeddings, left_n_tokens, right_n_tokens,
           W_tok_n, b_tok_n, W_tok_g, b_tok_g, W_tok_lin, b_tok_lin,
           attr_emb_left, attr_emb_right, W_ent_n, b_ent_n, W_ent_g, b_ent_g,
           W_ent_lin, b_ent_lin, empty_attr_res):
    return _run(left_embeddings, right_embeddings, left_n_tokens,
                right_n_tokens, W_tok_n, b_tok_n, W_tok_g, b_tok_g,
                W_tok_lin.reshape(D), attr_emb_left, attr_emb_right,
                W_ent_n, b_ent_n, W_ent_g, b_ent_g, W_ent_lin, b_ent_lin,
                empty_attr_res)


# speedup vs baseline: 4.7686x; 1.1636x over previous
import functools

import jax
import jax.numpy as jnp
from jax.experimental import pallas as pl
from jax.experimental.pallas import tpu as pltpu

D = 256
L = 256
R = 256
NATTR = 4
SEG = L // NATTR
TL = 32
NT = L // TL
NCHUNK = 4
CROWS = TL // NCHUNK
ENT = 2 * NATTR * D


def _fused_kernel(lt_ref, lf_ref, rf_ref, wall_ref, ball_ref,
                  wlt_ref, ael_ref, aer_ref, empty_ref,
                  wen_hbm, weg_hbm, ben_ref, beg_ref, wel_ref, bel_ref,
                  ln_ref, rn_ref,
                  out_ref, s_ref, xent_ref, wen_v, weg_v, sem_n, sem_g):
    i = pl.program_id(0)
    rt = rf_ref[...]

    @pl.when(i == 0)
    def _start_streams():
        pltpu.make_async_copy(wen_hbm, wen_v, sem_n).start()
        pltpu.make_async_copy(weg_hbm, weg_v, sem_g).start()

    for c in range(NCHUNK):
        lt = lt_ref[c * CROWS:(c + 1) * CROWS, :]
        x3 = jnp.abs(lt[:, None, :] - rt[None, :, :])
        x = x3.reshape(CROWS * R, D)
        y = jnp.dot(x, wall_ref[...], preferred_element_type=jnp.float32)
        yb = y + ball_ref[...]
        h = jax.nn.relu(yb[:, :D])
        g = jax.nn.sigmoid(yb[:, D:])
        hw = g * (h - x) + x
        s_ref[pl.ds(i * TL + c * CROWS, CROWS), :] = jnp.sum(
            hw * wlt_ref[...], axis=1).reshape(CROWS, R)

    @pl.when(i == NT - 1)
    def _finalize():
        lf = lf_ref[...]
        S = s_ref[...]

        iota_lr = jax.lax.broadcasted_iota(jnp.int32, (L, R), 1)
        mx = jnp.max(S, axis=1, keepdims=True)
        idx = jnp.min(jnp.where(S == mx, iota_lr, R), axis=1, keepdims=True)
        ohl = (iota_lr == idx).astype(jnp.float32)
        lcm = jnp.abs(lf - jnp.dot(ohl, rt,
                                   preferred_element_type=jnp.float32))

        St = S.T
        iota_rl = jax.lax.broadcasted_iota(jnp.int32, (R, L), 1)
        mx2 = jnp.max(St, axis=1, keepdims=True)
        idx2 = jnp.min(jnp.where(St == mx2, iota_rl, L), axis=1,
                       keepdims=True)
        ohr = (iota_rl == idx2).astype(jnp.float32)
        rcm = jnp.abs(rt - jnp.dot(ohr, lf,
                                   preferred_element_type=jnp.float32))

        def attr_rows(tok_all, emb_ref, cm, n_ref, base):
            for att in range(NATTR):
                tok = tok_all[att * SEG:(att + 1) * SEG, :]
                logits = jnp.sum(tok * emb_ref[att:att + 1, :], axis=1,
                                 keepdims=True)
                e = jnp.exp(logits - jnp.max(logits))
                w = e / jnp.sum(e)
                seg = cm[att * SEG:(att + 1) * SEG, :]
                val = jnp.sum(w * seg, axis=0, keepdims=True)
                val = jnp.where(n_ref[att] == 0, empty_ref[...], val)
                xent_ref[:, pl.ds((base + att) * D, D)] = val

        attr_rows(lf, ael_ref, lcm, ln_ref, 0)
        attr_rows(rt, aer_ref, rcm, rn_ref, NATTR)

        pltpu.make_async_copy(wen_hbm, wen_v, sem_n).wait()
        pltpu.make_async_copy(weg_hbm, weg_v, sem_g).wait()
        xa = xent_ref[...]
        eh = jax.nn.relu(
            jnp.dot(xa, wen_v[...], preferred_element_type=jnp.float32)
            + ben_ref[...])
        eg = jax.nn.sigmoid(
            jnp.dot(xa, weg_v[...], preferred_element_type=jnp.float32)
            + beg_ref[...])
        ehw = eg * (eh - xa) + xa
        lin = jnp.dot(ehw, wel_ref[...],
                      preferred_element_type=jnp.float32) + bel_ref[...]
        e = jnp.exp(lin - jnp.max(lin))
        out_ref[...] = e / jnp.sum(e)


@functools.partial(jax.jit, static_argnames=("interpret",))
def _run(left, right, ln, rn, wtn, btn, wtg, btg, wtl, ael, aer,
         wen, ben, weg, beg, wel, bel, empty, interpret=False):
    f32 = jnp.float32
    const = lambda shape: pl.BlockSpec(shape, lambda i: (0, 0))
    out = pl.pallas_call(
        _fused_kernel,
        grid=(NT,),
        in_specs=[
            pl.BlockSpec((TL, D), lambda i: (i, 0)),
            const((L, D)),
            const((R, D)),
            const((D, 2 * D)),
            const((1, 2 * D)),
            const((1, D)),
            const((NATTR, D)), const((NATTR, D)),
            const((1, D)),
            pl.BlockSpec(memory_space=pl.ANY),
            pl.BlockSpec(memory_space=pl.ANY),
            const((1, ENT)),
            const((1, ENT)),
            const((ENT, 2)),
            const((1, 2)),
            pl.BlockSpec(memory_space=pltpu.SMEM),
            pl.BlockSpec(memory_space=pltpu.SMEM),
        ],
        out_specs=const((1, 2)),
        out_shape=jax.ShapeDtypeStruct((1, 2), f32),
        scratch_shapes=[
            pltpu.VMEM((L, R), f32),
            pltpu.VMEM((1, ENT), f32),
            pltpu.VMEM((ENT, ENT), f32),
            pltpu.VMEM((ENT, ENT), f32),
            pltpu.SemaphoreType.DMA,
            pltpu.SemaphoreType.DMA,
        ],
        interpret=interpret,
    )(left, left, right,
      jnp.concatenate([wtn, wtg], axis=1),
      jnp.concatenate([btn, btg]).reshape(1, 2 * D),
      wtl.reshape(1, D), ael, aer, empty.reshape(1, D),
      wen, weg, ben.reshape(1, ENT), beg.reshape(1, ENT),
      wel, bel.reshape(1, 2), ln, rn)
    return out.reshape(-1)


def kernel(left_embeddings, right_embeddings, left_n_tokens, right_n_tokens,
           W_tok_n, b_tok_n, W_tok_g, b_tok_g, W_tok_lin, b_tok_lin,
           attr_emb_left, attr_emb_right, W_ent_n, b_ent_n, W_ent_g, b_ent_g,
           W_ent_lin, b_ent_lin, empty_attr_res):
    return _run(left_embeddings, right_embeddings, left_n_tokens,
                right_n_tokens, W_tok_n, b_tok_n, W_tok_g, b_tok_g,
                W_tok_lin.reshape(D), attr_emb_left, attr_emb_right,
                W_ent_n, b_ent_n, W_ent_g, b_ent_g, W_ent_lin, b_ent_lin,
                empty_attr_res)


# speedup vs baseline: 5.1278x; 1.0753x over previous
import functools

import jax
import jax.numpy as jnp
from jax.experimental import pallas as pl
from jax.experimental.pallas import tpu as pltpu

D = 256
L = 256
R = 256
NATTR = 4
SEG = L // NATTR
TL = 64
NT = L // TL
NCHUNK = 8
CROWS = TL // NCHUNK
ENT = 2 * NATTR * D


def _fused_kernel(lt_ref, lf_ref, rf_ref, wall_ref, ball_ref,
                  wlt_ref, ael_ref, aer_ref, empty_ref,
                  wen_hbm, weg_hbm, ben_ref, beg_ref, wel_ref, bel_ref,
                  ln_ref, rn_ref,
                  out_ref, s_ref, xent_ref, wen_v, weg_v, sem_n, sem_g):
    i = pl.program_id(0)
    rt = rf_ref[...]

    @pl.when(i == 0)
    def _start_streams():
        pltpu.make_async_copy(wen_hbm, wen_v, sem_n).start()
        pltpu.make_async_copy(weg_hbm, weg_v, sem_g).start()

    for c in range(NCHUNK):
        lt = lt_ref[c * CROWS:(c + 1) * CROWS, :]
        x3 = jnp.abs(lt[:, None, :] - rt[None, :, :])
        x = x3.reshape(CROWS * R, D)
        y = jnp.dot(x, wall_ref[...], preferred_element_type=jnp.float32)
        yb = y + ball_ref[...]
        h = jax.nn.relu(yb[:, :D])
        g = jax.nn.sigmoid(yb[:, D:])
        hw = g * (h - x) + x
        s_ref[pl.ds(i * TL + c * CROWS, CROWS), :] = jnp.sum(
            hw * wlt_ref[...], axis=1).reshape(CROWS, R)

    @pl.when(i == NT - 1)
    def _finalize():
        lf = lf_ref[...]
        S = s_ref[...]

        iota_lr = jax.lax.broadcasted_iota(jnp.int32, (L, R), 1)
        mx = jnp.max(S, axis=1, keepdims=True)
        idx = jnp.min(jnp.where(S == mx, iota_lr, R), axis=1, keepdims=True)
        ohl = (iota_lr == idx).astype(jnp.float32)
        lcm = jnp.abs(lf - jnp.dot(ohl, rt,
                                   preferred_element_type=jnp.float32))

        St = S.T
        iota_rl = jax.lax.broadcasted_iota(jnp.int32, (R, L), 1)
        mx2 = jnp.max(St, axis=1, keepdims=True)
        idx2 = jnp.min(jnp.where(St == mx2, iota_rl, L), axis=1,
                       keepdims=True)
        ohr = (iota_rl == idx2).astype(jnp.float32)
        rcm = jnp.abs(rt - jnp.dot(ohr, lf,
                                   preferred_element_type=jnp.float32))

        def attr_rows(tok_all, emb_ref, cm, n_ref, base):
            for att in range(NATTR):
                tok = tok_all[att * SEG:(att + 1) * SEG, :]
                logits = jnp.sum(tok * emb_ref[att:att + 1, :], axis=1,
                                 keepdims=True)
                e = jnp.exp(logits - jnp.max(logits))
                w = e / jnp.sum(e)
                seg = cm[att * SEG:(att + 1) * SEG, :]
                val = jnp.sum(w * seg, axis=0, keepdims=True)
                val = jnp.where(n_ref[att] == 0, empty_ref[...], val)
                xent_ref[:, pl.ds((base + att) * D, D)] = val

        attr_rows(lf, ael_ref, lcm, ln_ref, 0)
        attr_rows(rt, aer_ref, rcm, rn_ref, NATTR)

        pltpu.make_async_copy(wen_hbm, wen_v, sem_n).wait()
        pltpu.make_async_copy(weg_hbm, weg_v, sem_g).wait()
        xa = xent_ref[...]
        eh = jax.nn.relu(
            jnp.dot(xa, wen_v[...], preferred_element_type=jnp.float32)
            + ben_ref[...])
        eg = jax.nn.sigmoid(
            jnp.dot(xa, weg_v[...], preferred_element_type=jnp.float32)
            + beg_ref[...])
        ehw = eg * (eh - xa) + xa
        lin = jnp.dot(ehw, wel_ref[...],
                      preferred_element_type=jnp.float32) + bel_ref[...]
        e = jnp.exp(lin - jnp.max(lin))
        out_ref[...] = e / jnp.sum(e)


@functools.partial(jax.jit, static_argnames=("interpret",))
def _run(left, right, ln, rn, wtn, btn, wtg, btg, wtl, ael, aer,
         wen, ben, weg, beg, wel, bel, empty, interpret=False):
    f32 = jnp.float32
    const = lambda shape: pl.BlockSpec(shape, lambda i: (0, 0))
    out = pl.pallas_call(
        _fused_kernel,
        grid=(NT,),
        in_specs=[
            pl.BlockSpec((TL, D), lambda i: (i, 0)),
            const((L, D)),
            const((R, D)),
            const((D, 2 * D)),
            const((1, 2 * D)),
            const((1, D)),
            const((NATTR, D)), const((NATTR, D)),
            const((1, D)),
            pl.BlockSpec(memory_space=pl.ANY),
            pl.BlockSpec(memory_space=pl.ANY),
            const((1, ENT)),
            const((1, ENT)),
            const((ENT, 2)),
            const((1, 2)),
            pl.BlockSpec(memory_space=pltpu.SMEM),
            pl.BlockSpec(memory_space=pltpu.SMEM),
        ],
        out_specs=const((1, 2)),
        out_shape=jax.ShapeDtypeStruct((1, 2), f32),
        scratch_shapes=[
            pltpu.VMEM((L, R), f32),
            pltpu.VMEM((1, ENT), f32),
            pltpu.VMEM((ENT, ENT), f32),
            pltpu.VMEM((ENT, ENT), f32),
            pltpu.SemaphoreType.DMA,
            pltpu.SemaphoreType.DMA,
        ],
        interpret=interpret,
    )(left, left, right,
      jnp.concatenate([wtn, wtg], axis=1),
      jnp.concatenate([btn, btg]).reshape(1, 2 * D),
      wtl.reshape(1, D), ael, aer, empty.reshape(1, D),
      wen, weg, ben.reshape(1, ENT), beg.reshape(1, ENT),
      wel, bel.reshape(1, 2), ln, rn)
    return out.reshape(-1)


def kernel(left_embeddings, right_embeddings, left_n_tokens, right_n_tokens,
           W_tok_n, b_tok_n, W_tok_g, b_tok_g, W_tok_lin, b_tok_lin,
           attr_emb_left, attr_emb_right, W_ent_n, b_ent_n, W_ent_g, b_ent_g,
           W_ent_lin, b_ent_lin, empty_attr_res):
    return _run(left_embeddings, right_embeddings, left_n_tokens,
                right_n_tokens, W_tok_n, b_tok_n, W_tok_g, b_tok_g,
                W_tok_lin.reshape(D), attr_emb_left, attr_emb_right,
                W_ent_n, b_ent_n, W_ent_g, b_ent_g, W_ent_lin, b_ent_lin,
                empty_attr_res)


# speedup vs baseline: 5.2712x; 1.0280x over previous
import functools

import jax
import jax.numpy as jnp
from jax.experimental import pallas as pl
from jax.experimental.pallas import tpu as pltpu

D = 256
L = 256
R = 256
NATTR = 4
SEG = L // NATTR
TL = 64
NT = L // TL
NCHUNK = 8
CROWS = TL // NCHUNK
ENT = 2 * NATTR * D


def _fused_kernel(lt_ref, lf_ref, rf_ref, wtn_ref, wtg_ref, bn_ref, bg_ref,
                  wtl_ref, ael_ref, aer_ref, empty_ref,
                  wen_hbm, weg_hbm, ben_ref, beg_ref, wel_ref, bel_ref,
                  ln_ref, rn_ref,
                  out_ref, s_ref, xent_ref, wall_s, wlt_s,
                  wen_v, weg_v, sem_n, sem_g):
    i = pl.program_id(0)
    rt = rf_ref[...]

    @pl.when(i == 0)
    def _prologue():
        pltpu.make_async_copy(wen_hbm, wen_v, sem_n).start()
        pltpu.make_async_copy(weg_hbm, weg_v, sem_g).start()
        wall_s[:, :D] = wtn_ref[...]
        wall_s[:, D:] = wtg_ref[...]
        wlt_s[...] = wtl_ref[...].T

    bn = bn_ref[...]
    bg = bg_ref[...]
    wlt = wlt_s[...]

    for c in range(NCHUNK):
        lt = lt_ref[c * CROWS:(c + 1) * CROWS, :]
        x3 = jnp.abs(lt[:, None, :] - rt[None, :, :])
        x = x3.reshape(CROWS * R, D)
        y = jnp.dot(x, wall_s[...], preferred_element_type=jnp.float32)
        h = jax.nn.relu(y[:, :D] + bn)
        g = jax.nn.sigmoid(y[:, D:] + bg)
        hw = g * (h - x) + x
        s_ref[pl.ds(i * TL + c * CROWS, CROWS), :] = jnp.sum(
            hw * wlt, axis=1).reshape(CROWS, R)

    @pl.when(i == NT - 1)
    def _finalize():
        lf = lf_ref[...]
        S = s_ref[...]

        iota_lr = jax.lax.broadcasted_iota(jnp.int32, (L, R), 1)
        mx = jnp.max(S, axis=1, keepdims=True)
        idx = jnp.min(jnp.where(S == mx, iota_lr, R), axis=1, keepdims=True)
        ohl = (iota_lr == idx).astype(jnp.float32)
        lcm = jnp.abs(lf - jnp.dot(ohl, rt,
                                   preferred_element_type=jnp.float32))

        St = S.T
        iota_rl = jax.lax.broadcasted_iota(jnp.int32, (R, L), 1)
        mx2 = jnp.max(St, axis=1, keepdims=True)
        idx2 = jnp.min(jnp.where(St == mx2, iota_rl, L), axis=1,
                       keepdims=True)
        ohr = (iota_rl == idx2).astype(jnp.float32)
        rcm = jnp.abs(rt - jnp.dot(ohr, lf,
                                   preferred_element_type=jnp.float32))

        def attr_rows(tok_all, emb_ref, cm, n_ref, base):
            for att in range(NATTR):
                tok = tok_all[att * SEG:(att + 1) * SEG, :]
                logits = jnp.sum(tok * emb_ref[att:att + 1, :], axis=1,
                                 keepdims=True)
                e = jnp.exp(logits - jnp.max(logits))
                w = e / jnp.sum(e)
                seg = cm[att * SEG:(att + 1) * SEG, :]
                val = jnp.sum(w * seg, axis=0, keepdims=True)
                val = jnp.where(n_ref[att] == 0, empty_ref[...], val)
                xent_ref[:, pl.ds((base + att) * D, D)] = val

        attr_rows(lf, ael_ref, lcm, ln_ref, 0)
        attr_rows(rt, aer_ref, rcm, rn_ref, NATTR)

        pltpu.make_async_copy(wen_hbm, wen_v, sem_n).wait()
        pltpu.make_async_copy(weg_hbm, weg_v, sem_g).wait()
        xa = xent_ref[...]
        eh = jax.nn.relu(
            jnp.dot(xa, wen_v[...], preferred_element_type=jnp.float32)
            + ben_ref[...])
        eg = jax.nn.sigmoid(
            jnp.dot(xa, weg_v[...], preferred_element_type=jnp.float32)
            + beg_ref[...])
        ehw = eg * (eh - xa) + xa
        lin = jnp.dot(ehw, wel_ref[...],
                      preferred_element_type=jnp.float32) + bel_ref[...]
        e = jnp.exp(lin - jnp.max(lin))
        out_ref[...] = e / jnp.sum(e)


@functools.partial(jax.jit, static_argnames=("interpret",))
def _run(left, right, ln, rn, wtn, btn, wtg, btg, wtl, ael, aer,
         wen, ben, weg, beg, wel, bel, empty, interpret=False):
    f32 = jnp.float32
    const = lambda shape: pl.BlockSpec(shape, lambda i: (0, 0))
    out = pl.pallas_call(
        _fused_kernel,
        grid=(NT,),
        in_specs=[
            pl.BlockSpec((TL, D), lambda i: (i, 0)),
            const((L, D)),
            const((R, D)),
            const((D, D)), const((D, D)),
            const((1, D)), const((1, D)),
            const((D, 1)),
            const((NATTR, D)), const((NATTR, D)),
            const((1, D)),
            pl.BlockSpec(memory_space=pl.ANY),
            pl.BlockSpec(memory_space=pl.ANY),
            const((1, ENT)),
            const((1, ENT)),
            const((ENT, 2)),
            const((1, 2)),
            pl.BlockSpec(memory_space=pltpu.SMEM),
            pl.BlockSpec(memory_space=pltpu.SMEM),
        ],
        out_specs=const((1, 2)),
        out_shape=jax.ShapeDtypeStruct((1, 2), f32),
        scratch_shapes=[
            pltpu.VMEM((L, R), f32),
            pltpu.VMEM((1, ENT), f32),
            pltpu.VMEM((D, 2 * D), f32),
            pltpu.VMEM((1, D), f32),
            pltpu.VMEM((ENT, ENT), f32),
            pltpu.VMEM((ENT, ENT), f32),
            pltpu.SemaphoreType.DMA,
            pltpu.SemaphoreType.DMA,
        ],
        interpret=interpret,
    )(left, left, right, wtn, wtg,
      btn.reshape(1, D), btg.reshape(1, D),
      wtl, ael, aer, empty.reshape(1, D),
      wen, weg, ben.reshape(1, ENT), beg.reshape(1, ENT),
      wel, bel.reshape(1, 2), ln, rn)
    return out.reshape(-1)


def kernel(left_embeddings, right_embeddings, left_n_tokens, right_n_tokens,
           W_tok_n, b_tok_n, W_tok_g, b_tok_g, W_tok_lin, b_tok_lin,
           attr_emb_left, attr_emb_right, W_ent_n, b_ent_n, W_ent_g, b_ent_g,
           W_ent_lin, b_ent_lin, empty_attr_res):
    return _run(left_embeddings, right_embeddings, left_n_tokens,
                right_n_tokens, W_tok_n, b_tok_n, W_tok_g, b_tok_g,
                W_tok_lin, attr_emb_left, attr_emb_right,
                W_ent_n, b_ent_n, W_ent_g, b_ent_g, W_ent_lin, b_ent_lin,
                empty_attr_res)


# speedup vs baseline: 5.2727x; 1.0003x over previous
import functools

import jax
import jax.numpy as jnp
from jax.experimental import pallas as pl
from jax.experimental.pallas import tpu as pltpu

D = 256
L = 256
R = 256
NATTR = 4
SEG = L // NATTR
TL = 64
NT = L // TL
NCHUNK = 8
CROWS = TL // NCHUNK
ENT = 2 * NATTR * D


def _fused_kernel(lt_ref, lf_ref, rf_ref, wtn_ref, wtg_ref, bn_ref, bg_ref,
                  wtl_ref, ael_ref, aer_ref, empty_ref,
                  wen_hbm, weg_hbm, ben_ref, beg_ref, wel_ref, bel_ref,
                  ln_ref, rn_ref,
                  out_ref, s_ref, xent_ref, wall_s, wlt_s,
                  wen_v, weg_v, sem_n, sem_g):
    i = pl.program_id(0)
    rt = rf_ref[...]

    @pl.when(i == 0)
    def _prologue():
        pltpu.make_async_copy(wen_hbm, wen_v, sem_n).start()
        pltpu.make_async_copy(weg_hbm, weg_v, sem_g).start()
        wall_s[:, :D] = wtn_ref[...]
        wall_s[:, D:] = wtg_ref[...]
        wlt_s[...] = wtl_ref[...].T

    bn = bn_ref[...]
    bg = bg_ref[...]
    wlt = wlt_s[...]

    for c in range(NCHUNK):
        lt = lt_ref[c * CROWS:(c + 1) * CROWS, :]
        x3 = jnp.abs(lt[:, None, :] - rt[None, :, :])
        x = x3.reshape(CROWS * R, D)
        y = jnp.dot(x, wall_s[...], preferred_element_type=jnp.float32)
        h = jax.nn.relu(y[:, :D] + bn)
        g = jax.nn.sigmoid(y[:, D:] + bg)
        hw = g * (h - x) + x
        s_ref[pl.ds(i * TL + c * CROWS, CROWS), :] = jnp.sum(
            hw * wlt, axis=1).reshape(CROWS, R)

    @pl.when(i == NT - 1)
    def _finalize():
        lf = lf_ref[...]
        S = s_ref[...]

        iota_lr = jax.lax.broadcasted_iota(jnp.int32, (L, R), 1)
        mx = jnp.max(S, axis=1, keepdims=True)
        idx = jnp.min(jnp.where(S == mx, iota_lr, R), axis=1, keepdims=True)
        ohl = (iota_lr == idx).astype(jnp.float32)
        lcm = jnp.abs(lf - jnp.dot(ohl, rt,
                                   preferred_element_type=jnp.float32))

        St = S.T
        iota_rl = jax.lax.broadcasted_iota(jnp.int32, (R, L), 1)
        mx2 = jnp.max(St, axis=1, keepdims=True)
        idx2 = jnp.min(jnp.where(St == mx2, iota_rl, L), axis=1,
                       keepdims=True)
        ohr = (iota_rl == idx2).astype(jnp.float32)
        rcm = jnp.abs(rt - jnp.dot(ohr, lf,
                                   preferred_element_type=jnp.float32))

        def attr_rows(tok_all, emb_ref, cm, n_ref, base):
            for att in range(NATTR):
                tok = tok_all[att * SEG:(att + 1) * SEG, :]
                logits = jnp.sum(tok * emb_ref[att:att + 1, :], axis=1,
                                 keepdims=True)
                e = jnp.exp(logits - jnp.max(logits))
                w = e / jnp.sum(e)
                seg = cm[att * SEG:(att + 1) * SEG, :]
                val = jnp.sum(w * seg, axis=0, keepdims=True)
                val = jnp.where(n_ref[att] == 0, empty_ref[...], val)
                xent_ref[:, pl.ds((base + att) * D, D)] = val

        attr_rows(lf, ael_ref, lcm, ln_ref, 0)
        attr_rows(rt, aer_ref, rcm, rn_ref, NATTR)

        pltpu.make_async_copy(wen_hbm, wen_v, sem_n).wait()
        xa = xent_ref[...]
        eh = jax.nn.relu(
            jnp.dot(xa, wen_v[...], preferred_element_type=jnp.float32)
            + ben_ref[...])
        pltpu.make_async_copy(weg_hbm, weg_v, sem_g).wait()
        eg = jax.nn.sigmoid(
            jnp.dot(xa, weg_v[...], preferred_element_type=jnp.float32)
            + beg_ref[...])
        ehw = eg * (eh - xa) + xa
        lin = jnp.dot(ehw, wel_ref[...],
                      preferred_element_type=jnp.float32) + bel_ref[...]
        e = jnp.exp(lin - jnp.max(lin))
        out_ref[...] = e / jnp.sum(e)


@functools.partial(jax.jit, static_argnames=("interpret",))
def _run(left, right, ln, rn, wtn, btn, wtg, btg, wtl, ael, aer,
         wen, ben, weg, beg, wel, bel, empty, interpret=False):
    f32 = jnp.float32
    const = lambda shape: pl.BlockSpec(shape, lambda i: (0, 0))
    out = pl.pallas_call(
        _fused_kernel,
        grid=(NT,),
        in_specs=[
            pl.BlockSpec((TL, D), lambda i: (i, 0)),
            const((L, D)),
            const((R, D)),
            const((D, D)), const((D, D)),
            const((1, D)), const((1, D)),
            const((D, 1)),
            const((NATTR, D)), const((NATTR, D)),
            const((1, D)),
            pl.BlockSpec(memory_space=pl.ANY),
            pl.BlockSpec(memory_space=pl.ANY),
            const((1, ENT)),
            const((1, ENT)),
            const((ENT, 2)),
            const((1, 2)),
            pl.BlockSpec(memory_space=pltpu.SMEM),
            pl.BlockSpec(memory_space=pltpu.SMEM),
        ],
        out_specs=const((1, 2)),
        out_shape=jax.ShapeDtypeStruct((1, 2), f32),
        scratch_shapes=[
            pltpu.VMEM((L, R), f32),
            pltpu.VMEM((1, ENT), f32),
            pltpu.VMEM((D, 2 * D), f32),
            pltpu.VMEM((1, D), f32),
            pltpu.VMEM((ENT, ENT), f32),
            pltpu.VMEM((ENT, ENT), f32),
            pltpu.SemaphoreType.DMA,
            pltpu.SemaphoreType.DMA,
        ],
        interpret=interpret,
    )(left, left, right, wtn, wtg,
      btn.reshape(1, D), btg.reshape(1, D),
      wtl, ael, aer, empty.reshape(1, D),
      wen, weg, ben.reshape(1, ENT), beg.reshape(1, ENT),
      wel, bel.reshape(1, 2), ln, rn)
    return out.reshape(-1)


def kernel(left_embeddings, right_embeddings, left_n_tokens, right_n_tokens,
           W_tok_n, b_tok_n, W_tok_g, b_tok_g, W_tok_lin, b_tok_lin,
           attr_emb_left, attr_emb_right, W_ent_n, b_ent_n, W_ent_g, b_ent_g,
           W_ent_lin, b_ent_lin, empty_attr_res):
    return _run(left_embeddings, right_embeddings, left_n_tokens,
                right_n_tokens, W_tok_n, b_tok_n, W_tok_g, b_tok_g,
                W_tok_lin, attr_emb_left, attr_emb_right,
                W_ent_n, b_ent_n, W_ent_g, b_ent_g, W_ent_lin, b_ent_lin,
                empty_attr_res)


# speedup vs baseline: 5.2832x; 1.0020x over previous
import functools

import jax
import jax.numpy as jnp
from jax.experimental import pallas as pl
from jax.experimental.pallas import tpu as pltpu

D = 256
L = 256
R = 256
NATTR = 4
SEG = L // NATTR
TL = 64
NT = L // TL
NCHUNK = 8
CROWS = TL // NCHUNK
ENT = 2 * NATTR * D


def _fused_kernel(lt_ref, lf_ref, rf_ref, wtn_ref, wtg_ref, bn_ref, bg_ref,
                  wtl_ref, ael_ref, aer_ref, empty_ref,
                  wen_hbm, weg_hbm, ben_ref, beg_ref, wel_ref, bel_ref,
                  ln_ref, rn_ref,
                  out_ref, s_ref, wall_s, wlt_s,
                  wen_v, weg_v, sem_n, sem_g):
    i = pl.program_id(0)
    rt = rf_ref[...]

    @pl.when(i == 0)
    def _prologue():
        pltpu.make_async_copy(wen_hbm, wen_v, sem_n).start()
        pltpu.make_async_copy(weg_hbm, weg_v, sem_g).start()
        wall_s[:, :D] = wtn_ref[...]
        wall_s[:, D:] = wtg_ref[...]
        wlt_s[...] = wtl_ref[...].T

    bn = bn_ref[...]
    bg = bg_ref[...]
    wlt = wlt_s[...]

    for c in range(NCHUNK):
        lt = lt_ref[c * CROWS:(c + 1) * CROWS, :]
        x3 = jnp.abs(lt[:, None, :] - rt[None, :, :])
        x = x3.reshape(CROWS * R, D)
        y = jnp.dot(x, wall_s[...], preferred_element_type=jnp.float32)
        h = jax.nn.relu(y[:, :D] + bn)
        g = jax.nn.sigmoid(y[:, D:] + bg)
        hw = g * (h - x) + x
        s_ref[pl.ds(i * TL + c * CROWS, CROWS), :] = jnp.sum(
            hw * wlt, axis=1).reshape(CROWS, R)

    @pl.when(i == NT - 1)
    def _finalize():
        lf = lf_ref[...]
        S = s_ref[...]

        def attr_rows(tok_all, emb_ref, cm, n_ref):
            vals = []
            for att in range(NATTR):
                tok = tok_all[att * SEG:(att + 1) * SEG, :]
                logits = jnp.sum(tok * emb_ref[att:att + 1, :], axis=1,
                                 keepdims=True)
                e = jnp.exp(logits - jnp.max(logits))
                w = e / jnp.sum(e)
                seg = cm[att * SEG:(att + 1) * SEG, :]
                val = jnp.sum(w * seg, axis=0, keepdims=True)
                vals.append(jnp.where(n_ref[att] == 0, empty_ref[...], val))
            return jnp.concatenate(vals, axis=1)

        iota_lr = jax.lax.broadcasted_iota(jnp.int32, (L, R), 1)
        mx = jnp.max(S, axis=1, keepdims=True)
        idx = jnp.min(jnp.where(S == mx, iota_lr, R), axis=1, keepdims=True)
        ohl = (iota_lr == idx).astype(jnp.float32)
        lcm = jnp.abs(lf - jnp.dot(ohl, rt,
                                   preferred_element_type=jnp.float32))
        xl = attr_rows(lf, ael_ref, lcm, ln_ref)

        pltpu.make_async_copy(wen_hbm, wen_v, sem_n).wait()
        pltpu.make_async_copy(weg_hbm, weg_v, sem_g).wait()
        HALF = ENT // 2
        ehp = jnp.dot(xl, wen_v[:HALF, :],
                      preferred_element_type=jnp.float32)
        egp = jnp.dot(xl, weg_v[:HALF, :],
                      preferred_element_type=jnp.float32)

        St = S.T
        iota_rl = jax.lax.broadcasted_iota(jnp.int32, (R, L), 1)
        mx2 = jnp.max(St, axis=1, keepdims=True)
        idx2 = jnp.min(jnp.where(St == mx2, iota_rl, L), axis=1,
                       keepdims=True)
        ohr = (iota_rl == idx2).astype(jnp.float32)
        rcm = jnp.abs(rt - jnp.dot(ohr, lf,
                                   preferred_element_type=jnp.float32))
        xr = attr_rows(rt, aer_ref, rcm, rn_ref)

        xa = jnp.concatenate([xl, xr], axis=1)
        eh = jax.nn.relu(
            ehp + jnp.dot(xr, wen_v[HALF:, :],
                          preferred_element_type=jnp.float32) + ben_ref[...])
        eg = jax.nn.sigmoid(
            egp + jnp.dot(xr, weg_v[HALF:, :],
                          preferred_element_type=jnp.float32) + beg_ref[...])
        ehw = eg * (eh - xa) + xa
        lin = jnp.dot(ehw, wel_ref[...],
                      preferred_element_type=jnp.float32) + bel_ref[...]
        e = jnp.exp(lin - jnp.max(lin))
        out_ref[...] = e / jnp.sum(e)


@functools.partial(jax.jit, static_argnames=("interpret",))
def _run(left, right, ln, rn, wtn, btn, wtg, btg, wtl, ael, aer,
         wen, ben, weg, beg, wel, bel, empty, interpret=False):
    f32 = jnp.float32
    const = lambda shape: pl.BlockSpec(shape, lambda i: (0, 0))
    out = pl.pallas_call(
        _fused_kernel,
        grid=(NT,),
        in_specs=[
            pl.BlockSpec((TL, D), lambda i: (i, 0)),
            const((L, D)),
            const((R, D)),
            const((D, D)), const((D, D)),
            const((1, D)), const((1, D)),
            const((D, 1)),
            const((NATTR, D)), const((NATTR, D)),
            const((1, D)),
            pl.BlockSpec(memory_space=pl.ANY),
            pl.BlockSpec(memory_space=pl.ANY),
            const((1, ENT)),
            const((1, ENT)),
            const((ENT, 2)),
            const((1, 2)),
            pl.BlockSpec(memory_space=pltpu.SMEM),
            pl.BlockSpec(memory_space=pltpu.SMEM),
        ],
        out_specs=const((1, 2)),
        out_shape=jax.ShapeDtypeStruct((1, 2), f32),
        scratch_shapes=[
            pltpu.VMEM((L, R), f32),
            pltpu.VMEM((D, 2 * D), f32),
            pltpu.VMEM((1, D), f32),
            pltpu.VMEM((ENT, ENT), f32),
            pltpu.VMEM((ENT, ENT), f32),
            pltpu.SemaphoreType.DMA,
            pltpu.SemaphoreType.DMA,
        ],
        interpret=interpret,
    )(left, left, right, wtn, wtg,
      btn.reshape(1, D), btg.reshape(1, D),
      wtl, ael, aer, empty.reshape(1, D),
      wen, weg, ben.reshape(1, ENT), beg.reshape(1, ENT),
      wel, bel.reshape(1, 2), ln, rn)
    return out.reshape(-1)


def kernel(left_embeddings, right_embeddings, left_n_tokens, right_n_tokens,
           W_tok_n, b_tok_n, W_tok_g, b_tok_g, W_tok_lin, b_tok_lin,
           attr_emb_left, attr_emb_right, W_ent_n, b_ent_n, W_ent_g, b_ent_g,
           W_ent_lin, b_ent_lin, empty_attr_res):
    return _run(left_embeddings, right_embeddings, left_n_tokens,
                right_n_tokens, W_tok_n, b_tok_n, W_tok_g, b_tok_g,
                W_tok_lin, attr_emb_left, attr_emb_right,
                W_ent_n, b_ent_n, W_ent_g, b_ent_g, W_ent_lin, b_ent_lin,
                empty_attr_res)


# speedup vs baseline: 5.3902x; 1.0203x over previous
import functools

import jax
import jax.numpy as jnp
from jax.experimental import pallas as pl
from jax.experimental.pallas import tpu as pltpu

D = 256
L = 256
R = 256
NATTR = 4
SEG = L // NATTR
TL = 128
NT = L // TL
NCHUNK = 8
CROWS = TL // NCHUNK
ENT = 2 * NATTR * D


def _fused_kernel(lt_ref, lf_ref, rf_ref, wtn_ref, wtg_ref, bn_ref, bg_ref,
                  wtl_ref, ael_ref, aer_ref, empty_ref,
                  wen_hbm, weg_hbm, ben_ref, beg_ref, wel_ref, bel_ref,
                  ln_ref, rn_ref,
                  out_ref, s_ref, wall_s, wlt_s,
                  wen_v, weg_v, sem_n, sem_g):
    i = pl.program_id(0)
    rt = rf_ref[...]

    @pl.when(i == 0)
    def _prologue():
        pltpu.make_async_copy(wen_hbm, wen_v, sem_n).start()
        pltpu.make_async_copy(weg_hbm, weg_v, sem_g).start()
        wall_s[:, :D] = wtn_ref[...]
        wall_s[:, D:] = wtg_ref[...]
        wlt_s[...] = wtl_ref[...].T

    bn = bn_ref[...]
    bg = bg_ref[...]
    wlt = wlt_s[...]

    for c in range(NCHUNK):
        lt = lt_ref[c * CROWS:(c + 1) * CROWS, :]
        x3 = jnp.abs(lt[:, None, :] - rt[None, :, :])
        x = x3.reshape(CROWS * R, D)
        y = jnp.dot(x, wall_s[...], preferred_element_type=jnp.float32)
        h = jax.nn.relu(y[:, :D] + bn)
        g = jax.nn.sigmoid(y[:, D:] + bg)
        hw = g * (h - x) + x
        s_ref[pl.ds(i * TL + c * CROWS, CROWS), :] = jnp.sum(
            hw * wlt, axis=1).reshape(CROWS, R)

    @pl.when(i == NT - 1)
    def _finalize():
        lf = lf_ref[...]
        S = s_ref[...]

        def attr_rows(tok_all, emb_ref, cm, n_ref):
            vals = []
            for att in range(NATTR):
                tok = tok_all[att * SEG:(att + 1) * SEG, :]
                logits = jnp.sum(tok * emb_ref[att:att + 1, :], axis=1,
                                 keepdims=True)
                e = jnp.exp(logits - jnp.max(logits))
                w = e / jnp.sum(e)
                seg = cm[att * SEG:(att + 1) * SEG, :]
                val = jnp.sum(w * seg, axis=0, keepdims=True)
                vals.append(jnp.where(n_ref[att] == 0, empty_ref[...], val))
            return jnp.concatenate(vals, axis=1)

        iota_lr = jax.lax.broadcasted_iota(jnp.int32, (L, R), 1)
        mx = jnp.max(S, axis=1, keepdims=True)
        idx = jnp.min(jnp.where(S == mx, iota_lr, R), axis=1, keepdims=True)
        ohl = (iota_lr == idx).astype(jnp.float32)
        lcm = jnp.abs(lf - jnp.dot(ohl, rt,
                                   preferred_element_type=jnp.float32))
        xl = attr_rows(lf, ael_ref, lcm, ln_ref)

        pltpu.make_async_copy(wen_hbm, wen_v, sem_n).wait()
        pltpu.make_async_copy(weg_hbm, weg_v, sem_g).wait()
        HALF = ENT // 2
        ehp = jnp.dot(xl, wen_v[:HALF, :],
                      preferred_element_type=jnp.float32)
        egp = jnp.dot(xl, weg_v[:HALF, :],
                      preferred_element_type=jnp.float32)

        St = S.T
        iota_rl = jax.lax.broadcasted_iota(jnp.int32, (R, L), 1)
        mx2 = jnp.max(St, axis=1, keepdims=True)
        idx2 = jnp.min(jnp.where(St == mx2, iota_rl, L), axis=1,
                       keepdims=True)
        ohr = (iota_rl == idx2).astype(jnp.float32)
        rcm = jnp.abs(rt - jnp.dot(ohr, lf,
                                   preferred_element_type=jnp.float32))
        xr = attr_rows(rt, aer_ref, rcm, rn_ref)

        xa = jnp.concatenate([xl, xr], axis=1)
        eh = jax.nn.relu(
            ehp + jnp.dot(xr, wen_v[HALF:, :],
                          preferred_element_type=jnp.float32) + ben_ref[...])
        eg = jax.nn.sigmoid(
            egp + jnp.dot(xr, weg_v[HALF:, :],
                          preferred_element_type=jnp.float32) + beg_ref[...])
        ehw = eg * (eh - xa) + xa
        lin = jnp.dot(ehw, wel_ref[...],
                      preferred_element_type=jnp.float32) + bel_ref[...]
        e = jnp.exp(lin - jnp.max(lin))
        out_ref[...] = e / jnp.sum(e)


@functools.partial(jax.jit, static_argnames=("interpret",))
def _run(left, right, ln, rn, wtn, btn, wtg, btg, wtl, ael, aer,
         wen, ben, weg, beg, wel, bel, empty, interpret=False):
    f32 = jnp.float32
    const = lambda shape: pl.BlockSpec(shape, lambda i: (0, 0))
    out = pl.pallas_call(
        _fused_kernel,
        grid=(NT,),
        in_specs=[
            pl.BlockSpec((TL, D), lambda i: (i, 0)),
            const((L, D)),
            const((R, D)),
            const((D, D)), const((D, D)),
            const((1, D)), const((1, D)),
            const((D, 1)),
            const((NATTR, D)), const((NATTR, D)),
            const((1, D)),
            pl.BlockSpec(memory_space=pl.ANY),
            pl.BlockSpec(memory_space=pl.ANY),
            const((1, ENT)),
            const((1, ENT)),
            const((ENT, 2)),
            const((1, 2)),
            pl.BlockSpec(memory_space=pltpu.SMEM),
            pl.BlockSpec(memory_space=pltpu.SMEM),
        ],
        out_specs=const((1, 2)),
        out_shape=jax.ShapeDtypeStruct((1, 2), f32),
        scratch_shapes=[
            pltpu.VMEM((L, R), f32),
            pltpu.VMEM((D, 2 * D), f32),
            pltpu.VMEM((1, D), f32),
            pltpu.VMEM((ENT, ENT), f32),
            pltpu.VMEM((ENT, ENT), f32),
            pltpu.SemaphoreType.DMA,
            pltpu.SemaphoreType.DMA,
        ],
        interpret=interpret,
    )(left, left, right, wtn, wtg,
      btn.reshape(1, D), btg.reshape(1, D),
      wtl, ael, aer, empty.reshape(1, D),
      wen, weg, ben.reshape(1, ENT), beg.reshape(1, ENT),
      wel, bel.reshape(1, 2), ln, rn)
    return out.reshape(-1)


def kernel(left_embeddings, right_embeddings, left_n_tokens, right_n_tokens,
           W_tok_n, b_tok_n, W_tok_g, b_tok_g, W_tok_lin, b_tok_lin,
           attr_emb_left, attr_emb_right, W_ent_n, b_ent_n, W_ent_g, b_ent_g,
           W_ent_lin, b_ent_lin, empty_attr_res):
    return _run(left_embeddings, right_embeddings, left_n_tokens,
                right_n_tokens, W_tok_n, b_tok_n, W_tok_g, b_tok_g,
                W_tok_lin, attr_emb_left, attr_emb_right,
                W_ent_n, b_ent_n, W_ent_g, b_ent_g, W_ent_lin, b_ent_lin,
                empty_attr_res)


# speedup vs baseline: 5.4313x; 1.0076x over previous
import functools

import jax
import jax.numpy as jnp
from jax.experimental import pallas as pl
from jax.experimental.pallas import tpu as pltpu

D = 256
L = 256
R = 256
NATTR = 4
SEG = L // NATTR
TL = 128
NT = L // TL
NCHUNK = 8
CROWS = TL // NCHUNK
ENT = 2 * NATTR * D


def _fused_kernel(lf_ref, rf_ref, wtn_ref, wtg_ref, bn_ref, bg_ref,
                  wtl_ref, ael_ref, aer_ref, empty_ref,
                  wen_hbm, weg_hbm, ben_ref, beg_ref, wel_ref, bel_ref,
                  ln_ref, rn_ref,
                  out_ref, s_ref, wall_s, wlt_s,
                  wen_v, weg_v, sem_n, sem_g):
    i = pl.program_id(0)
    rt = rf_ref[...]

    @pl.when(i == 0)
    def _prologue():
        pltpu.make_async_copy(wen_hbm, wen_v, sem_n).start()
        pltpu.make_async_copy(weg_hbm, weg_v, sem_g).start()
        wall_s[:, :D] = wtn_ref[...]
        wall_s[:, D:] = wtg_ref[...]
        wlt_s[...] = wtl_ref[...].T

    bn = bn_ref[...]
    bg = bg_ref[...]
    wlt = wlt_s[...]

    for c in range(NCHUNK):
        lt = lf_ref[pl.ds(i * TL + c * CROWS, CROWS), :]
        x3 = jnp.abs(lt[:, None, :] - rt[None, :, :])
        x = x3.reshape(CROWS * R, D)
        y = jnp.dot(x, wall_s[...], preferred_element_type=jnp.float32)
        h = jax.nn.relu(y[:, :D] + bn)
        g = jax.nn.sigmoid(y[:, D:] + bg)
        hw = g * (h - x) + x
        s_ref[pl.ds(i * TL + c * CROWS, CROWS), :] = jnp.sum(
            hw * wlt, axis=1).reshape(CROWS, R)

    @pl.when(i == NT - 1)
    def _finalize():
        lf = lf_ref[...]
        S = s_ref[...]

        def attr_rows(tok_all, emb_ref, cm, n_ref):
            vals = []
            for att in range(NATTR):
                tok = tok_all[att * SEG:(att + 1) * SEG, :]
                logits = jnp.sum(tok * emb_ref[att:att + 1, :], axis=1,
                                 keepdims=True)
                e = jnp.exp(logits - jnp.max(logits))
                w = e / jnp.sum(e)
                seg = cm[att * SEG:(att + 1) * SEG, :]
                val = jnp.sum(w * seg, axis=0, keepdims=True)
                vals.append(jnp.where(n_ref[att] == 0, empty_ref[...], val))
            return jnp.concatenate(vals, axis=1)

        iota_lr = jax.lax.broadcasted_iota(jnp.int32, (L, R), 1)
        mx = jnp.max(S, axis=1, keepdims=True)
        idx = jnp.min(jnp.where(S == mx, iota_lr, R), axis=1, keepdims=True)
        ohl = (iota_lr == idx).astype(jnp.float32)
        lcm = jnp.abs(lf - jnp.dot(ohl, rt,
                                   preferred_element_type=jnp.float32))
        xl = attr_rows(lf, ael_ref, lcm, ln_ref)

        pltpu.make_async_copy(wen_hbm, wen_v, sem_n).wait()
        pltpu.make_async_copy(weg_hbm, weg_v, sem_g).wait()
        HALF = ENT // 2
        ehp = jnp.dot(xl, wen_v[:HALF, :],
                      preferred_element_type=jnp.float32)
        egp = jnp.dot(xl, weg_v[:HALF, :],
                      preferred_element_type=jnp.float32)

        St = S.T
        iota_rl = jax.lax.broadcasted_iota(jnp.int32, (R, L), 1)
        mx2 = jnp.max(St, axis=1, keepdims=True)
        idx2 = jnp.min(jnp.where(St == mx2, iota_rl, L), axis=1,
                       keepdims=True)
        ohr = (iota_rl == idx2).astype(jnp.float32)
        rcm = jnp.abs(rt - jnp.dot(ohr, lf,
                                   preferred_element_type=jnp.float32))
        xr = attr_rows(rt, aer_ref, rcm, rn_ref)

        xa = jnp.concatenate([xl, xr], axis=1)
        eh = jax.nn.relu(
            ehp + jnp.dot(xr, wen_v[HALF:, :],
                          preferred_element_type=jnp.float32) + ben_ref[...])
        eg = jax.nn.sigmoid(
            egp + jnp.dot(xr, weg_v[HALF:, :],
                          preferred_element_type=jnp.float32) + beg_ref[...])
        ehw = eg * (eh - xa) + xa
        lin = jnp.dot(ehw, wel_ref[...],
                      preferred_element_type=jnp.float32) + bel_ref[...]
        e = jnp.exp(lin - jnp.max(lin))
        out_ref[...] = e / jnp.sum(e)


@functools.partial(jax.jit, static_argnames=("interpret",))
def _run(left, right, ln, rn, wtn, btn, wtg, btg, wtl, ael, aer,
         wen, ben, weg, beg, wel, bel, empty, interpret=False):
    f32 = jnp.float32
    const = lambda shape: pl.BlockSpec(shape, lambda i: (0, 0))
    out = pl.pallas_call(
        _fused_kernel,
        grid=(NT,),
        in_specs=[
            const((L, D)),
            const((R, D)),
            const((D, D)), const((D, D)),
            const((1, D)), const((1, D)),
            const((D, 1)),
            const((NATTR, D)), const((NATTR, D)),
            const((1, D)),
            pl.BlockSpec(memory_space=pl.ANY),
            pl.BlockSpec(memory_space=pl.ANY),
            const((1, ENT)),
            const((1, ENT)),
            const((ENT, 2)),
            const((1, 2)),
            pl.BlockSpec(memory_space=pltpu.SMEM),
            pl.BlockSpec(memory_space=pltpu.SMEM),
        ],
        out_specs=const((1, 2)),
        out_shape=jax.ShapeDtypeStruct((1, 2), f32),
        scratch_shapes=[
            pltpu.VMEM((L, R), f32),
            pltpu.VMEM((D, 2 * D), f32),
            pltpu.VMEM((1, D), f32),
            pltpu.VMEM((ENT, ENT), f32),
            pltpu.VMEM((ENT, ENT), f32),
            pltpu.SemaphoreType.DMA,
            pltpu.SemaphoreType.DMA,
        ],
        interpret=interpret,
    )(left, right, wtn, wtg,
      btn.reshape(1, D), btg.reshape(1, D),
      wtl, ael, aer, empty.reshape(1, D),
      wen, weg, ben.reshape(1, ENT), beg.reshape(1, ENT),
      wel, bel.reshape(1, 2), ln, rn)
    return out.reshape(-1)


def kernel(left_embeddings, right_embeddings, left_n_tokens, right_n_tokens,
           W_tok_n, b_tok_n, W_tok_g, b_tok_g, W_tok_lin, b_tok_lin,
           attr_emb_left, attr_emb_right, W_ent_n, b_ent_n, W_ent_g, b_ent_g,
           W_ent_lin, b_ent_lin, empty_attr_res):
    return _run(left_embeddings, right_embeddings, left_n_tokens,
                right_n_tokens, W_tok_n, b_tok_n, W_tok_g, b_tok_g,
                W_tok_lin, attr_emb_left, attr_emb_right,
                W_ent_n, b_ent_n, W_ent_g, b_ent_g, W_ent_lin, b_ent_lin,
                empty_attr_res)
